# baseline (speedup 1.0000x reference)
import jax
import jax.numpy as jnp
from jax import lax
from jax.experimental import pallas as pl
from jax.experimental.pallas import tpu as pltpu
from jax.experimental.pallas import tpu_sc as plsc

_NUM_ENTRIES = 1000
_DEPTH = 12
_NUM_PER_SLOT = 4
_EMBED_DIM = 768
_B = 1024
_K = 2
_BK = _B * _K
_NC = 2
_NS = 16
_NW = _NC * _NS
_PER_W = _BK // _NW
_BLK = 8
_NBLK = _PER_W // _BLK
_NBUF = 4
_LANES = 16


def _sc_body(idx_hbm, tab_hbm, out4_hbm, idx_v, idx_all, buf0, buf1, buf2,
             buf3, gsem0, gsem1, gsem2, gsem3, ssem0, ssem1, ssem2, ssem3):
    wid = lax.axis_index("s") * _NC + lax.axis_index("c")
    base = wid * _PER_W
    pltpu.sync_copy(idx_hbm.at[pl.ds(base, _PER_W)], idx_v)
    bufs = (buf0, buf1, buf2, buf3)
    gsems = (gsem0, gsem1, gsem2, gsem3)
    ssems = (ssem0, ssem1, ssem2, ssem3)

    for d in range(_DEPTH):
        for c in range(_PER_W // _LANES):
            idx_all[pl.ds(d * _PER_W + c * _LANES, _LANES)] = (
                idx_v[pl.ds(c * _LANES, _LANES)] * _DEPTH + d)

    nstage = _NBLK * _DEPTH
    gathers = [None] * nstage
    scatters = [None] * nstage

    def start_gather(t):
        j, d = divmod(t, _DEPTH)
        gathers[t] = pltpu.async_copy(
            tab_hbm.at[idx_all.at[pl.ds(d * _PER_W + j * _BLK, _BLK)]],
            bufs[t % _NBUF], gsems[t % _NBUF])

    def start_scatter(t):
        j, d = divmod(t, _DEPTH)
        b0 = (base + j * _BLK) // _K
        scatters[t] = pltpu.async_copy(
            bufs[t % _NBUF].reshape(_BLK // _K, _K * _NUM_PER_SLOT, _EMBED_DIM),
            out4_hbm.at[d, pl.ds(b0, _BLK // _K), :, :], ssems[t % _NBUF])

    for t in range(nstage):
        if t >= _NBUF:
            scatters[t - _NBUF].wait()
        start_gather(t)
        if t >= 1:
            gathers[t - 1].wait()
            start_scatter(t - 1)
    gathers[nstage - 1].wait()
    start_scatter(nstage - 1)
    for t in range(nstage - _NBUF + 1, nstage):
        scatters[t].wait()


_mesh = plsc.VectorSubcoreMesh(
    core_axis_name="c", subcore_axis_name="s",
    num_cores=_NC, num_subcores=_NS)

_sc_call = pl.kernel(
    _sc_body,
    out_type=jax.ShapeDtypeStruct((_DEPTH, _B, _K * _NUM_PER_SLOT, _EMBED_DIM),
                                  jnp.float32),
    mesh=_mesh,
    scratch_types=[
        pltpu.VMEM((_PER_W,), jnp.int32),
        pltpu.VMEM((_DEPTH * _PER_W,), jnp.int32),
        pltpu.VMEM((_BLK, _NUM_PER_SLOT, _EMBED_DIM), jnp.float32),
        pltpu.VMEM((_BLK, _NUM_PER_SLOT, _EMBED_DIM), jnp.float32),
        pltpu.VMEM((_BLK, _NUM_PER_SLOT, _EMBED_DIM), jnp.float32),
        pltpu.VMEM((_BLK, _NUM_PER_SLOT, _EMBED_DIM), jnp.float32),
        pltpu.SemaphoreType.DMA,
        pltpu.SemaphoreType.DMA,
        pltpu.SemaphoreType.DMA,
        pltpu.SemaphoreType.DMA,
        pltpu.SemaphoreType.DMA,
        pltpu.SemaphoreType.DMA,
        pltpu.SemaphoreType.DMA,
        pltpu.SemaphoreType.DMA,
    ],
)


@jax.jit
def kernel(indices, prompts):
    flat = indices.reshape(-1)
    tab = prompts.reshape(_NUM_ENTRIES * _DEPTH, _NUM_PER_SLOT, _EMBED_DIM)
    return _sc_call(flat, tab)


# speedup vs baseline: 1.2788x; 1.2788x over previous
import jax
import jax.numpy as jnp
from jax import lax
from jax.experimental import pallas as pl
from jax.experimental.pallas import tpu as pltpu
from jax.experimental.pallas import tpu_sc as plsc

_NUM_ENTRIES = 1000
_DEPTH = 12
_NUM_PER_SLOT = 4
_EMBED_DIM = 768
_B = 1024
_K = 2
_BK = _B * _K
_NC = 2
_NS = 16
_DL = _DEPTH // _NC
_NTH = 6
_THW = _EMBED_DIM // _NTH
_PER_T = _BK // _NS
_BLK = 16
_NBLK = _PER_T // _BLK
_NBUF = 4
_NSTAGE = _DL * _NTH
_E_PER_TILE = 64


def _sc_body(idx_hbm, prompts_hbm, out4_hbm, idx_v, buf0, buf1, buf2, buf3,
             cache0, cache1, stsem, gsem, ssem0, ssem1, ssem2, ssem3):
    c = lax.axis_index("c")
    s = lax.axis_index("s")
    pltpu.sync_copy(idx_hbm.at[pl.ds(s * _PER_T, _PER_T)], idx_v)
    bufs = (buf0, buf1, buf2, buf3)
    ssems = (ssem0, ssem1, ssem2, ssem3)
    caches = (cache0, cache1)
    nstage = _DL * _NTH
    e0 = jnp.minimum(s * _E_PER_TILE, _NUM_ENTRIES - _E_PER_TILE)

    def stage_refs(t):
        dl, r = divmod(t, _NTH)
        d = _DL * c + dl
        src = prompts_hbm.at[pl.ds(e0, _E_PER_TILE), d, :,
                             pl.ds(r * _THW, _THW)]
        dst = caches[t % 2].at[pl.ds(e0, _E_PER_TILE), :, :]
        return src, dst

    scatters = [None] * (nstage * _NBLK)

    def consume(t):
        dl, r = divmod(t, _NTH)
        d = _DL * c + dl
        p = caches[t % 2]
        for jb in range(_NBLK):
            u = t * _NBLK + jb
            q = u % _NBUF
            if u >= _NBUF:
                scatters[u - _NBUF].wait()
            lanes = idx_v[pl.ds(jb * _BLK, _BLK)]
            pltpu.async_copy(p.at[lanes], bufs[q], gsem).wait()
            b0 = s * (_PER_T // _K) + jb * (_BLK // _K)
            scatters[u] = pltpu.async_copy(
                bufs[q].reshape(_BLK // _K, _K * _NUM_PER_SLOT, _THW),
                out4_hbm.at[d, pl.ds(b0, _BLK // _K), :,
                            pl.ds(r * _THW, _THW)],
                ssems[q])

    src, dst = stage_refs(0)
    pltpu.async_copy(src, dst, stsem).wait()
    plsc.subcore_barrier()
    for t in range(nstage):
        if t + 1 < nstage:
            src, dst = stage_refs(t + 1)
            nxt = pltpu.async_copy(src, dst, stsem)
        consume(t)
        if t + 1 < nstage:
            nxt.wait()
            plsc.subcore_barrier()
    for u in range(nstage * _NBLK - _NBUF, nstage * _NBLK):
        scatters[u].wait()


_mesh = plsc.VectorSubcoreMesh(
    core_axis_name="c", subcore_axis_name="s",
    num_cores=_NC, num_subcores=_NS)

_sc_call = pl.kernel(
    _sc_body,
    out_type=jax.ShapeDtypeStruct((_DEPTH, _B, _K * _NUM_PER_SLOT, _EMBED_DIM),
                                  jnp.float32),
    mesh=_mesh,
    scratch_types=[
        pltpu.VMEM((_PER_T,), jnp.int32),
        pltpu.VMEM((_BLK, _NUM_PER_SLOT, _THW), jnp.float32),
        pltpu.VMEM((_BLK, _NUM_PER_SLOT, _THW), jnp.float32),
        pltpu.VMEM((_BLK, _NUM_PER_SLOT, _THW), jnp.float32),
        pltpu.VMEM((_BLK, _NUM_PER_SLOT, _THW), jnp.float32),
        pltpu.VMEM_SHARED((_NUM_ENTRIES, _NUM_PER_SLOT, _THW), jnp.float32),
        pltpu.VMEM_SHARED((_NUM_ENTRIES, _NUM_PER_SLOT, _THW), jnp.float32),
        pltpu.SemaphoreType.DMA,
        pltpu.SemaphoreType.DMA,
        pltpu.SemaphoreType.DMA,
        pltpu.SemaphoreType.DMA,
        pltpu.SemaphoreType.DMA,
        pltpu.SemaphoreType.DMA,
    ],
)


@jax.jit
def kernel(indices, prompts):
    flat = indices.reshape(-1)
    return _sc_call(flat, prompts)


# speedup vs baseline: 1.3002x; 1.0167x over previous
import jax
import jax.numpy as jnp
from jax import lax
from jax.experimental import pallas as pl
from jax.experimental.pallas import tpu as pltpu
from jax.experimental.pallas import tpu_sc as plsc

_NUM_ENTRIES = 1000
_DEPTH = 12
_NUM_PER_SLOT = 4
_EMBED_DIM = 768
_B = 1024
_K = 2
_BK = _B * _K
_NC = 2
_NS = 16
_DL = _DEPTH // _NC
_NTH = 6
_THW = _EMBED_DIM // _NTH
_PER_T = _BK // _NS
_BLK = 32
_NBLK = _PER_T // _BLK
_NBUF = 4
_NSTAGE = _DL * _NTH
_E_PER_TILE = 64


def _sc_body(idx_hbm, prompts_hbm, out4_hbm, idx_v, buf0, buf1, buf2, buf3,
             cache0, cache1, stsem, gsem, ssem0, ssem1, ssem2, ssem3):
    c = lax.axis_index("c")
    s = lax.axis_index("s")
    pltpu.sync_copy(idx_hbm.at[pl.ds(s * _PER_T, _PER_T)], idx_v)
    bufs = (buf0, buf1, buf2, buf3)
    ssems = (ssem0, ssem1, ssem2, ssem3)
    caches = (cache0, cache1)
    nstage = _DL * _NTH
    e0 = jnp.minimum(s * _E_PER_TILE, _NUM_ENTRIES - _E_PER_TILE)

    def stage_refs(t):
        dl, r = divmod(t, _NTH)
        d = _DL * c + dl
        src = prompts_hbm.at[pl.ds(e0, _E_PER_TILE), d, :,
                             pl.ds(r * _THW, _THW)]
        dst = caches[t % 2].at[pl.ds(e0, _E_PER_TILE), :, :]
        return src, dst

    scatters = [None] * (nstage * _NBLK)

    def consume(t):
        dl, r = divmod(t, _NTH)
        d = _DL * c + dl
        p = caches[t % 2]
        for jb in range(_NBLK):
            u = t * _NBLK + jb
            q = u % _NBUF
            if u >= _NBUF:
                scatters[u - _NBUF].wait()
            lanes = idx_v.at[pl.ds(jb * _BLK, _BLK)]
            pltpu.async_copy(p.at[lanes], bufs[q], gsem).wait()
            b0 = s * (_PER_T // _K) + jb * (_BLK // _K)
            scatters[u] = pltpu.async_copy(
                bufs[q].reshape(_BLK // _K, _K * _NUM_PER_SLOT, _THW),
                out4_hbm.at[d, pl.ds(b0, _BLK // _K), :,
                            pl.ds(r * _THW, _THW)],
                ssems[q])

    src, dst = stage_refs(0)
    pltpu.async_copy(src, dst, stsem).wait()
    plsc.subcore_barrier()
    for t in range(nstage):
        if t + 1 < nstage:
            src, dst = stage_refs(t + 1)
            nxt = pltpu.async_copy(src, dst, stsem)
        consume(t)
        if t + 1 < nstage:
            nxt.wait()
            plsc.subcore_barrier()
    for u in range(nstage * _NBLK - _NBUF, nstage * _NBLK):
        scatters[u].wait()


_mesh = plsc.VectorSubcoreMesh(
    core_axis_name="c", subcore_axis_name="s",
    num_cores=_NC, num_subcores=_NS)

_sc_call = pl.kernel(
    _sc_body,
    out_type=jax.ShapeDtypeStruct((_DEPTH, _B, _K * _NUM_PER_SLOT, _EMBED_DIM),
                                  jnp.float32),
    mesh=_mesh,
    scratch_types=[
        pltpu.VMEM((_PER_T,), jnp.int32),
        pltpu.VMEM((_BLK, _NUM_PER_SLOT, _THW), jnp.float32),
        pltpu.VMEM((_BLK, _NUM_PER_SLOT, _THW), jnp.float32),
        pltpu.VMEM((_BLK, _NUM_PER_SLOT, _THW), jnp.float32),
        pltpu.VMEM((_BLK, _NUM_PER_SLOT, _THW), jnp.float32),
        pltpu.VMEM_SHARED((_NUM_ENTRIES, _NUM_PER_SLOT, _THW), jnp.float32),
        pltpu.VMEM_SHARED((_NUM_ENTRIES, _NUM_PER_SLOT, _THW), jnp.float32),
        pltpu.SemaphoreType.DMA,
        pltpu.SemaphoreType.DMA,
        pltpu.SemaphoreType.DMA,
        pltpu.SemaphoreType.DMA,
        pltpu.SemaphoreType.DMA,
        pltpu.SemaphoreType.DMA,
    ],
)


@jax.jit
def kernel(indices, prompts):
    flat = indices.reshape(-1)
    return _sc_call(flat, prompts)
